# initial kernel scaffold (unmeasured)
import jax
import jax.numpy as jnp
from jax import lax
from jax.experimental import pallas as pl
from jax.experimental.pallas import tpu as pltpu

N_DEV = 4


def kernel(x, w_mat):
    m_per, k = x.shape
    _, n_per = w_mat.shape

    def body(x_ref, w_ref, out_ref, comm_ref, send_sems, recv_sems):
        my = lax.axis_index("i")
        left = (my - 1) % N_DEV
        right = (my + 1) % N_DEV

        barrier_sem = pltpu.get_barrier_semaphore()
        for nbr in [left, right]:
            pl.semaphore_signal(
                barrier_sem, inc=1,
                device_id=(nbr,), device_id_type=pl.DeviceIdType.MESH,
            )
        pl.semaphore_wait(barrier_sem, 2)

        acc = jnp.dot(x_ref[...], w_ref[...], preferred_element_type=jnp.float32)
        out_ref[pl.ds(my * m_per, m_per), :] = acc * jax.nn.sigmoid(acc)

        for h in range(N_DEV - 1):
            src = x_ref if h == 0 else comm_ref.at[h - 1]
            rdma = pltpu.make_async_remote_copy(
                src_ref=src,
                dst_ref=comm_ref.at[h],
                send_sem=send_sems.at[h],
                recv_sem=recv_sems.at[h],
                device_id=(right,),
                device_id_type=pl.DeviceIdType.MESH,
            )
            rdma.start()
            rdma.wait()
            origin = (my - h - 1) % N_DEV
            chunk = comm_ref[h]
            acc = jnp.dot(chunk, w_ref[...], preferred_element_type=jnp.float32)
            out_ref[pl.ds(origin * m_per, m_per), :] = acc * jax.nn.sigmoid(acc)

    return pl.pallas_call(
        body,
        out_shape=jax.ShapeDtypeStruct((N_DEV * m_per, n_per), jnp.float32),
        in_specs=[
            pl.BlockSpec(memory_space=pltpu.VMEM),
            pl.BlockSpec(memory_space=pltpu.VMEM),
        ],
        out_specs=pl.BlockSpec(memory_space=pltpu.VMEM),
        scratch_shapes=[
            pltpu.VMEM((N_DEV - 1, m_per, k), jnp.float32),
            pltpu.SemaphoreType.DMA((N_DEV - 1,)),
            pltpu.SemaphoreType.DMA((N_DEV - 1,)),
        ],
        compiler_params=pltpu.CompilerParams(collective_id=0),
    )(x, w_mat)


# baseline (device time: 571764 ns/iter reference)
import jax
import jax.numpy as jnp
from jax import lax
from jax.experimental import pallas as pl
from jax.experimental.pallas import tpu as pltpu

N_DEV = 4


def kernel(x, w_mat):
    m_per, k = x.shape
    _, n_per = w_mat.shape

    def body(x_hbm, w_ref, out_ref, comm_ref, send_sems, recv_sems,
             credit_sem, local_sem):
        my = lax.axis_index("i")
        left = (my - 1) % N_DEV
        right = (my + 1) % N_DEV

        def silu_store(slot, origin):
            acc = jnp.dot(comm_ref[slot], w_ref[...],
                          preferred_element_type=jnp.float32)
            out_ref[pl.ds(origin * m_per, m_per), :] = acc * jax.nn.sigmoid(acc)

        cp = pltpu.make_async_copy(x_hbm, comm_ref.at[0], local_sem)
        cp.start()
        cp.wait()

        barrier_sem = pltpu.get_barrier_semaphore()
        for nbr in [left, right]:
            pl.semaphore_signal(
                barrier_sem, inc=1,
                device_id=(nbr,), device_id_type=pl.DeviceIdType.MESH,
            )
        pl.semaphore_wait(barrier_sem, 2)

        r0 = pltpu.make_async_remote_copy(
            src_ref=comm_ref.at[0], dst_ref=comm_ref.at[1],
            send_sem=send_sems.at[0], recv_sem=recv_sems.at[0],
            device_id=(right,), device_id_type=pl.DeviceIdType.MESH,
        )
        r0.start()
        silu_store(0, my)
        r0.wait_send()
        pl.semaphore_signal(credit_sem, inc=1, device_id=(left,),
                            device_id_type=pl.DeviceIdType.MESH)
        r0.wait_recv()

        pl.semaphore_wait(credit_sem, 1)
        r1 = pltpu.make_async_remote_copy(
            src_ref=comm_ref.at[1], dst_ref=comm_ref.at[0],
            send_sem=send_sems.at[1], recv_sem=recv_sems.at[1],
            device_id=(right,), device_id_type=pl.DeviceIdType.MESH,
        )
        r1.start()
        silu_store(1, left)
        r1.wait_send()
        pl.semaphore_signal(credit_sem, inc=1, device_id=(left,),
                            device_id_type=pl.DeviceIdType.MESH)
        r1.wait_recv()

        pl.semaphore_wait(credit_sem, 1)
        r2 = pltpu.make_async_remote_copy(
            src_ref=comm_ref.at[0], dst_ref=comm_ref.at[1],
            send_sem=send_sems.at[2], recv_sem=recv_sems.at[2],
            device_id=(right,), device_id_type=pl.DeviceIdType.MESH,
        )
        r2.start()
        silu_store(0, (my - 2) % N_DEV)
        r2.wait_send()
        r2.wait_recv()
        silu_store(1, right)

    return pl.pallas_call(
        body,
        out_shape=jax.ShapeDtypeStruct((N_DEV * m_per, n_per), jnp.float32),
        in_specs=[
            pl.BlockSpec(memory_space=pl.ANY),
            pl.BlockSpec(memory_space=pltpu.VMEM),
        ],
        out_specs=pl.BlockSpec(memory_space=pltpu.VMEM),
        scratch_shapes=[
            pltpu.VMEM((2, m_per, k), jnp.float32),
            pltpu.SemaphoreType.DMA((N_DEV - 1,)),
            pltpu.SemaphoreType.DMA((N_DEV - 1,)),
            pltpu.SemaphoreType.REGULAR,
            pltpu.SemaphoreType.DMA,
        ],
        compiler_params=pltpu.CompilerParams(
            collective_id=0,
            vmem_limit_bytes=56 * 1024 * 1024,
        ),
    )(x, w_mat)


# device time: 213586 ns/iter; 2.6770x vs baseline; 2.6770x over previous
import jax
import jax.numpy as jnp
from jax import lax
from jax.experimental import pallas as pl
from jax.experimental.pallas import tpu as pltpu

N_DEV = 4


def kernel(x, w_mat):
    m_per, k = x.shape
    _, n_per = w_mat.shape
    n_half = n_per // 2

    def body(x_ref, w_ref, out_hbm, w_bufs, res_buf,
             w_send, w_recv, res_send, res_recv, local_sem):
        my = lax.axis_index("i")
        left = (my - 1) % N_DEV
        right = (my + 1) % N_DEV
        diag = (my + 2) % N_DEV

        def gemm_silu(w_in, slot):
            acc = jnp.dot(x_ref[...], w_in,
                          preferred_element_type=jnp.float32)
            res_buf[slot] = acc * jax.nn.sigmoid(acc)

        def row_block(dev):
            return out_hbm.at[pl.ds(dev * m_per, m_per), :]

        barrier_sem = pltpu.get_barrier_semaphore()
        for nbr in [left, right, diag]:
            pl.semaphore_signal(
                barrier_sem, inc=1,
                device_id=(nbr,), device_id_type=pl.DeviceIdType.MESH,
            )
        pl.semaphore_wait(barrier_sem, 3)

        d_w_r = pltpu.make_async_remote_copy(
            src_ref=w_ref, dst_ref=w_bufs.at[0],
            send_sem=w_send.at[0], recv_sem=w_recv.at[0],
            device_id=(right,), device_id_type=pl.DeviceIdType.MESH,
        )
        d_w_r.start()
        d_w_l = pltpu.make_async_remote_copy(
            src_ref=w_ref, dst_ref=w_bufs.at[1],
            send_sem=w_send.at[1], recv_sem=w_recv.at[1],
            device_id=(left,), device_id_type=pl.DeviceIdType.MESH,
        )
        d_w_l.start()

        gemm_silu(w_ref[...], 3)
        cp_own = pltpu.make_async_copy(res_buf.at[3], row_block(my), local_sem)
        cp_own.start()

        d_w_r.wait_recv()
        d_fwd_r = pltpu.make_async_remote_copy(
            src_ref=w_bufs.at[0, :, :n_half],
            dst_ref=w_bufs.at[2, :, :n_half],
            send_sem=w_send.at[2], recv_sem=w_recv.at[2],
            device_id=(right,), device_id_type=pl.DeviceIdType.MESH,
        )
        d_fwd_r.start()
        gemm_silu(w_bufs[0], 1)
        s_res_l = pltpu.make_async_remote_copy(
            src_ref=res_buf.at[1], dst_ref=row_block(my),
            send_sem=res_send.at[1], recv_sem=res_recv.at[1],
            device_id=(left,), device_id_type=pl.DeviceIdType.MESH,
        )
        s_res_l.start()

        d_w_l.wait_recv()
        d_fwd_l = pltpu.make_async_remote_copy(
            src_ref=w_bufs.at[1, :, n_half:],
            dst_ref=w_bufs.at[2, :, n_half:],
            send_sem=w_send.at[3], recv_sem=w_recv.at[3],
            device_id=(left,), device_id_type=pl.DeviceIdType.MESH,
        )
        d_fwd_l.start()
        gemm_silu(w_bufs[1], 0)
        s_res_r = pltpu.make_async_remote_copy(
            src_ref=res_buf.at[0], dst_ref=row_block(my),
            send_sem=res_send.at[0], recv_sem=res_recv.at[0],
            device_id=(right,), device_id_type=pl.DeviceIdType.MESH,
        )
        s_res_r.start()

        d_fwd_r.wait_recv()
        d_fwd_l.wait_recv()
        gemm_silu(w_bufs[2], 2)
        s_res_d = pltpu.make_async_remote_copy(
            src_ref=res_buf.at[2], dst_ref=row_block(my),
            send_sem=res_send.at[2], recv_sem=res_recv.at[2],
            device_id=(diag,), device_id_type=pl.DeviceIdType.MESH,
        )
        s_res_d.start()

        d_w_r.wait_send()
        d_w_l.wait_send()
        d_fwd_r.wait_send()
        d_fwd_l.wait_send()
        s_res_l.wait_send()
        s_res_r.wait_send()
        s_res_d.wait_send()
        cp_own.wait()

        for src_dev, sem_idx in [(left, 0), (right, 1), (diag, 2)]:
            recv = pltpu.make_async_remote_copy(
                src_ref=res_buf.at[2], dst_ref=row_block(src_dev),
                send_sem=res_send.at[2], recv_sem=res_recv.at[sem_idx],
                device_id=(my,), device_id_type=pl.DeviceIdType.MESH,
            )
            recv.wait_recv()

    return pl.pallas_call(
        body,
        out_shape=jax.ShapeDtypeStruct((N_DEV * m_per, n_per), jnp.float32),
        in_specs=[
            pl.BlockSpec(memory_space=pltpu.VMEM),
            pl.BlockSpec(memory_space=pltpu.VMEM),
        ],
        out_specs=pl.BlockSpec(memory_space=pl.ANY),
        scratch_shapes=[
            pltpu.VMEM((3, k, n_per), jnp.float32),
            pltpu.VMEM((4, m_per, n_per), jnp.float32),
            pltpu.SemaphoreType.DMA((4,)),
            pltpu.SemaphoreType.DMA((4,)),
            pltpu.SemaphoreType.DMA((3,)),
            pltpu.SemaphoreType.DMA((3,)),
            pltpu.SemaphoreType.DMA,
        ],
        compiler_params=pltpu.CompilerParams(
            collective_id=0,
            vmem_limit_bytes=62 * 1024 * 1024,
        ),
    )(x, w_mat)


# device time: 122387 ns/iter; 4.6718x vs baseline; 1.7452x over previous
import jax
import jax.numpy as jnp
from jax import lax
from jax.experimental import pallas as pl
from jax.experimental.pallas import tpu as pltpu

N_DEV = 4


def kernel(x, w_mat):
    m_per, k = x.shape
    _, n_per = w_mat.shape
    n_half = n_per // 2
    m_strip = m_per // 4

    def body(x_hbm, w_ref, out_ref, w_bufs, res_bf, res_in, x_bf, w_bf,
             x_stage, w_send, w_recv, res_send, res_recv, stage_sem):
        my = lax.axis_index("i")
        left = (my - 1) % N_DEV
        right = (my + 1) % N_DEV
        diag = (my + 2) % N_DEV

        def gemm_silu(w_in, slot):
            acc = jnp.dot(x_bf[...], w_in,
                          preferred_element_type=jnp.float32)
            res_bf[slot] = (acc * jax.nn.sigmoid(acc)).astype(jnp.bfloat16)

        w_bf[...] = w_ref[...].astype(jnp.bfloat16)
        barrier_sem = pltpu.get_barrier_semaphore()
        for nbr in [left, right, diag]:
            pl.semaphore_signal(
                barrier_sem, inc=1,
                device_id=(nbr,), device_id_type=pl.DeviceIdType.MESH,
            )
        pl.semaphore_wait(barrier_sem, 3)

        d_w_r = pltpu.make_async_remote_copy(
            src_ref=w_bf, dst_ref=w_bufs.at[0],
            send_sem=w_send.at[0], recv_sem=w_recv.at[0],
            device_id=(right,), device_id_type=pl.DeviceIdType.MESH,
        )
        d_w_r.start()
        d_w_l = pltpu.make_async_remote_copy(
            src_ref=w_bf, dst_ref=w_bufs.at[1],
            send_sem=w_send.at[1], recv_sem=w_recv.at[1],
            device_id=(left,), device_id_type=pl.DeviceIdType.MESH,
        )
        d_w_l.start()

        for s in range(4):
            cp = pltpu.make_async_copy(
                x_hbm.at[pl.ds(s * m_strip, m_strip), :],
                x_stage, stage_sem)
            cp.start()
            cp.wait()
            x_bf[pl.ds(s * m_strip, m_strip), :] = (
                x_stage[...].astype(jnp.bfloat16))

        acc = jnp.dot(x_bf[...], w_bf[...], preferred_element_type=jnp.float32)
        out_ref[pl.ds(my * m_per, m_per), :] = acc * jax.nn.sigmoid(acc)

        d_w_r.wait_recv()
        d_fwd_r = pltpu.make_async_remote_copy(
            src_ref=w_bufs.at[0, :, :n_half],
            dst_ref=w_bufs.at[2, :, :n_half],
            send_sem=w_send.at[2], recv_sem=w_recv.at[2],
            device_id=(right,), device_id_type=pl.DeviceIdType.MESH,
        )
        d_fwd_r.start()
        gemm_silu(w_bufs[0], 1)
        s_res_l = pltpu.make_async_remote_copy(
            src_ref=res_bf.at[1], dst_ref=res_in.at[1],
            send_sem=res_send.at[1], recv_sem=res_recv.at[1],
            device_id=(left,), device_id_type=pl.DeviceIdType.MESH,
        )
        s_res_l.start()

        d_w_l.wait_recv()
        d_fwd_l = pltpu.make_async_remote_copy(
            src_ref=w_bufs.at[1, :, n_half:],
            dst_ref=w_bufs.at[2, :, n_half:],
            send_sem=w_send.at[3], recv_sem=w_recv.at[3],
            device_id=(left,), device_id_type=pl.DeviceIdType.MESH,
        )
        d_fwd_l.start()
        gemm_silu(w_bufs[1], 0)
        s_res_r = pltpu.make_async_remote_copy(
            src_ref=res_bf.at[0], dst_ref=res_in.at[0],
            send_sem=res_send.at[0], recv_sem=res_recv.at[0],
            device_id=(right,), device_id_type=pl.DeviceIdType.MESH,
        )
        s_res_r.start()

        d_fwd_r.wait_recv()
        d_fwd_l.wait_recv()
        gemm_silu(w_bufs[2], 2)
        s_res_d = pltpu.make_async_remote_copy(
            src_ref=res_bf.at[2], dst_ref=res_in.at[2],
            send_sem=res_send.at[2], recv_sem=res_recv.at[2],
            device_id=(diag,), device_id_type=pl.DeviceIdType.MESH,
        )
        s_res_d.start()

        for src_dev, sem_idx in [(left, 0), (right, 1), (diag, 2)]:
            recv = pltpu.make_async_remote_copy(
                src_ref=res_bf.at[2], dst_ref=res_in.at[sem_idx],
                send_sem=res_send.at[2], recv_sem=res_recv.at[sem_idx],
                device_id=(my,), device_id_type=pl.DeviceIdType.MESH,
            )
            recv.wait_recv()
            out_ref[pl.ds(src_dev * m_per, m_per), :] = (
                res_in[sem_idx].astype(jnp.float32))

        d_w_r.wait_send()
        d_w_l.wait_send()
        d_fwd_r.wait_send()
        d_fwd_l.wait_send()
        s_res_l.wait_send()
        s_res_r.wait_send()
        s_res_d.wait_send()

    return pl.pallas_call(
        body,
        out_shape=jax.ShapeDtypeStruct((N_DEV * m_per, n_per), jnp.float32),
        in_specs=[
            pl.BlockSpec(memory_space=pl.ANY),
            pl.BlockSpec(memory_space=pltpu.VMEM),
        ],
        out_specs=pl.BlockSpec(memory_space=pltpu.VMEM),
        scratch_shapes=[
            pltpu.VMEM((3, k, n_per), jnp.bfloat16),
            pltpu.VMEM((3, m_per, n_per), jnp.bfloat16),
            pltpu.VMEM((3, m_per, n_per), jnp.bfloat16),
            pltpu.VMEM((m_per, k), jnp.bfloat16),
            pltpu.VMEM((k, n_per), jnp.bfloat16),
            pltpu.VMEM((m_strip, k), jnp.float32),
            pltpu.SemaphoreType.DMA((4,)),
            pltpu.SemaphoreType.DMA((4,)),
            pltpu.SemaphoreType.DMA((3,)),
            pltpu.SemaphoreType.DMA((3,)),
            pltpu.SemaphoreType.DMA,
        ],
        compiler_params=pltpu.CompilerParams(
            collective_id=0,
            vmem_limit_bytes=60 * 1024 * 1024,
        ),
    )(x, w_mat)


# device time: 122073 ns/iter; 4.6838x vs baseline; 1.0026x over previous
import jax
import jax.numpy as jnp
from jax import lax
from jax.experimental import pallas as pl
from jax.experimental.pallas import tpu as pltpu

N_DEV = 4


def kernel(x, w_mat):
    m_per, k = x.shape
    _, n_per = w_mat.shape
    n_half = n_per // 2
    m_strip = m_per // 4

    def body(x_hbm, w_ref, out_ref, w_bufs, res_bf, res_in, x_bf, w_bf,
             x_stage, w_send, w_recv, res_send, res_recv, stage_sem):
        my = lax.axis_index("i")
        left = (my - 1) % N_DEV
        right = (my + 1) % N_DEV
        diag = (my + 2) % N_DEV

        def gemm_silu(w_in, slot):
            acc = jnp.dot(x_bf[...], w_in,
                          preferred_element_type=jnp.float32)
            res_bf[slot] = (acc * jax.nn.sigmoid(acc)).astype(jnp.bfloat16)

        barrier_sem = pltpu.get_barrier_semaphore()
        for nbr in [left, right, diag]:
            pl.semaphore_signal(
                barrier_sem, inc=1,
                device_id=(nbr,), device_id_type=pl.DeviceIdType.MESH,
            )
        w_bf[...] = w_ref[...].astype(jnp.bfloat16)
        pl.semaphore_wait(barrier_sem, 3)

        d_w_r = pltpu.make_async_remote_copy(
            src_ref=w_bf, dst_ref=w_bufs.at[0],
            send_sem=w_send.at[0], recv_sem=w_recv.at[0],
            device_id=(right,), device_id_type=pl.DeviceIdType.MESH,
        )
        d_w_r.start()
        d_w_l = pltpu.make_async_remote_copy(
            src_ref=w_bf, dst_ref=w_bufs.at[1],
            send_sem=w_send.at[1], recv_sem=w_recv.at[1],
            device_id=(left,), device_id_type=pl.DeviceIdType.MESH,
        )
        d_w_l.start()

        for s in range(4):
            cp = pltpu.make_async_copy(
                x_hbm.at[pl.ds(s * m_strip, m_strip), :],
                x_stage, stage_sem)
            cp.start()
            cp.wait()
            x_bf[pl.ds(s * m_strip, m_strip), :] = (
                x_stage[...].astype(jnp.bfloat16))

        acc = jnp.dot(x_bf[...], w_bf[...], preferred_element_type=jnp.float32)
        out_ref[pl.ds(my * m_per, m_per), :] = acc * jax.nn.sigmoid(acc)

        d_w_r.wait_recv()
        d_fwd_r = pltpu.make_async_remote_copy(
            src_ref=w_bufs.at[0, :, :n_half],
            dst_ref=w_bufs.at[2, :, :n_half],
            send_sem=w_send.at[2], recv_sem=w_recv.at[2],
            device_id=(right,), device_id_type=pl.DeviceIdType.MESH,
        )
        d_fwd_r.start()
        gemm_silu(w_bufs[0], 1)
        s_res_l = pltpu.make_async_remote_copy(
            src_ref=res_bf.at[1], dst_ref=res_in.at[1],
            send_sem=res_send.at[1], recv_sem=res_recv.at[1],
            device_id=(left,), device_id_type=pl.DeviceIdType.MESH,
        )
        s_res_l.start()

        d_w_l.wait_recv()
        d_fwd_l = pltpu.make_async_remote_copy(
            src_ref=w_bufs.at[1, :, n_half:],
            dst_ref=w_bufs.at[2, :, n_half:],
            send_sem=w_send.at[3], recv_sem=w_recv.at[3],
            device_id=(left,), device_id_type=pl.DeviceIdType.MESH,
        )
        d_fwd_l.start()
        gemm_silu(w_bufs[1], 0)
        s_res_r = pltpu.make_async_remote_copy(
            src_ref=res_bf.at[0], dst_ref=res_in.at[0],
            send_sem=res_send.at[0], recv_sem=res_recv.at[0],
            device_id=(right,), device_id_type=pl.DeviceIdType.MESH,
        )
        s_res_r.start()

        d_fwd_r.wait_recv()
        d_fwd_l.wait_recv()
        gemm_silu(w_bufs[2], 2)
        s_res_d = pltpu.make_async_remote_copy(
            src_ref=res_bf.at[2], dst_ref=res_in.at[2],
            send_sem=res_send.at[2], recv_sem=res_recv.at[2],
            device_id=(diag,), device_id_type=pl.DeviceIdType.MESH,
        )
        s_res_d.start()

        for src_dev, sem_idx in [(left, 0), (right, 1), (diag, 2)]:
            recv = pltpu.make_async_remote_copy(
                src_ref=res_bf.at[2], dst_ref=res_in.at[sem_idx],
                send_sem=res_send.at[2], recv_sem=res_recv.at[sem_idx],
                device_id=(my,), device_id_type=pl.DeviceIdType.MESH,
            )
            recv.wait_recv()
            out_ref[pl.ds(src_dev * m_per, m_per), :] = (
                res_in[sem_idx].astype(jnp.float32))

        d_w_r.wait_send()
        d_w_l.wait_send()
        d_fwd_r.wait_send()
        d_fwd_l.wait_send()
        s_res_l.wait_send()
        s_res_r.wait_send()
        s_res_d.wait_send()

    return pl.pallas_call(
        body,
        out_shape=jax.ShapeDtypeStruct((N_DEV * m_per, n_per), jnp.float32),
        in_specs=[
            pl.BlockSpec(memory_space=pl.ANY),
            pl.BlockSpec(memory_space=pltpu.VMEM),
        ],
        out_specs=pl.BlockSpec(memory_space=pltpu.VMEM),
        scratch_shapes=[
            pltpu.VMEM((3, k, n_per), jnp.bfloat16),
            pltpu.VMEM((3, m_per, n_per), jnp.bfloat16),
            pltpu.VMEM((3, m_per, n_per), jnp.bfloat16),
            pltpu.VMEM((m_per, k), jnp.bfloat16),
            pltpu.VMEM((k, n_per), jnp.bfloat16),
            pltpu.VMEM((m_strip, k), jnp.float32),
            pltpu.SemaphoreType.DMA((4,)),
            pltpu.SemaphoreType.DMA((4,)),
            pltpu.SemaphoreType.DMA((3,)),
            pltpu.SemaphoreType.DMA((3,)),
            pltpu.SemaphoreType.DMA,
        ],
        compiler_params=pltpu.CompilerParams(
            collective_id=0,
            vmem_limit_bytes=60 * 1024 * 1024,
        ),
    )(x, w_mat)
